# baseline (speedup 1.0000x reference)
import functools

import jax
import jax.numpy as jnp
from jax import lax
from jax.experimental import pallas as pl
from jax.experimental.pallas import tpu as pltpu
from jax.experimental.pallas import tpu_sc as plsc

_SHIFT = 50000.0
_NC = 2
_NS = 16
_LANES = 16
_PAD_B = 256


def _sc_gather(x_pad, embedding):
    mesh = plsc.ScalarSubcoreMesh(axis_name="c", num_cores=_NC)
    per_core = _PAD_B // _NC

    @functools.partial(
        pl.kernel,
        mesh=mesh,
        out_type=jax.ShapeDtypeStruct((_PAD_B, embedding.shape[1]), jnp.float32),
        scratch_types=[
            pltpu.SMEM((_PAD_B,), jnp.float32),
            pltpu.SemaphoreType.DMA,
        ],
    )
    def k(x_hbm, emb_hbm, out_hbm, xs, sem):
        cid = lax.axis_index("c")
        base = cid * per_core
        pltpu.async_copy(x_hbm, xs, sem).wait()

        @pl.loop(0, per_core)
        def _(i):
            t = base + i
            idx = (xs[t] + _SHIFT).astype(jnp.int32)
            pltpu.async_copy(emb_hbm.at[pl.ds(idx, 1)], out_hbm.at[pl.ds(t, 1)], sem)

        @pl.loop(0, per_core)
        def _(i):
            pltpu.make_async_copy(
                emb_hbm.at[pl.ds(0, 1)], out_hbm.at[pl.ds(base, 1)], sem
            ).wait()

    return k(x_pad, embedding)


def _matvec(h, w, b, k_blk, tanh):
    kdim, n = w.shape
    nk = kdim // k_blk

    def body(h_ref, w_ref, b_ref, o_ref):
        i = pl.program_id(0)

        @pl.when(i == 0)
        def _():
            o_ref[...] = jnp.zeros_like(o_ref)

        o_ref[...] += jnp.dot(
            h_ref[...], w_ref[...], preferred_element_type=jnp.float32
        )

        @pl.when(i == nk - 1)
        def _():
            r = o_ref[...] + b_ref[...]
            o_ref[...] = jnp.tanh(r) if tanh else r

    return pl.pallas_call(
        body,
        grid=(nk,),
        in_specs=[
            pl.BlockSpec((1, k_blk), lambda i: (0, i)),
            pl.BlockSpec((k_blk, n), lambda i: (i, 0)),
            pl.BlockSpec((1, n), lambda i: (0, 0)),
        ],
        out_specs=pl.BlockSpec((1, n), lambda i: (0, 0)),
        out_shape=jax.ShapeDtypeStruct((1, n), jnp.float32),
    )(h, w, b)


def kernel(x, embedding, W0, b0, W1, b1, W2, b2):
    x_pad = jnp.concatenate([x, jnp.zeros((_PAD_B - x.shape[0],), x.dtype)])
    rows = _sc_gather(x_pad, embedding)
    h0 = rows.reshape(1, _PAD_B * embedding.shape[1])
    h1 = _matvec(h0, W0, b0.reshape(1, -1), 1280, True)
    h2 = _matvec(h1, W1, b1.reshape(1, -1), 512, True)
    out = _matvec(h2, W2, b2.reshape(1, -1), 512, False)
    return out.reshape(-1)
